# baseline (speedup 1.0000x reference)
import functools

import jax
import jax.numpy as jnp
from jax import lax
from jax.experimental import pallas as pl
from jax.experimental.pallas import tpu as pltpu
from jax.experimental.pallas import tpu_sc as plsc

N = 10000
E = 320000
D = 128
L = 2

NC = 2
NS = 16
NW = NC * NS
EPW = E // NW
K = 80
NCHUNK = EPW // K
NBUF = 4
EU = 20
NPAD = 10240
ROWS_PT = NPAD // NS


def _sc_agg_body(m_hbm, meta_hbm, out_hbm,
                 m0, m1, m2, m3, r0, r1, r2, r3, agg_sh,
                 ms0, ms1, ms2, ms3, gs0, gs1, gs2, gs3,
                 ss0, ss1, ss2, ss3):
    c = lax.axis_index("c")
    s = lax.axis_index("s")
    wid = c * NS + s
    meta_v = (m0, m1, m2, m3)
    rows = (r0, r1, r2, r3)
    msem = (ms0, ms1, ms2, ms3)
    gsem = (gs0, gs1, gs2, gs3)
    ssem = (ss0, ss1, ss2, ss3)

    def meta_issue(g, b):
        pltpu.async_copy(meta_hbm.at[wid, g], meta_v[b], msem[b])

    def meta_wait(b):
        pltpu.make_async_copy(meta_hbm.at[0, 0], meta_v[b], msem[b]).wait()

    def gather_issue(b):
        pltpu.async_copy(m_hbm.at[meta_v[b].at[0]], rows[b], gsem[b])

    def gather_wait(b):
        pltpu.make_async_copy(m_hbm.at[meta_v[b].at[0]], rows[b],
                              gsem[b]).wait()

    def scatter_issue(b):
        pltpu.async_copy(rows[b], agg_sh.at[meta_v[b].at[1]], ssem[b],
                         add=True)

    def scatter_wait(b):
        pltpu.make_async_copy(rows[b], agg_sh.at[meta_v[b].at[1]],
                              ssem[b]).wait()

    def scale(b):
        def blk(i, carry):
            base = i * EU
            for j in range(EU):
                e = base + j
                ai = plsc.load_gather(
                    meta_v[b],
                    [jnp.full((16,), 2, dtype=jnp.int32),
                     jnp.full((16,), e, dtype=jnp.int32)])
                av = plsc.bitcast(ai, jnp.float32)
                for f in range(D // 16):
                    rows[b][e, pl.ds(f * 16, 16)] = (
                        rows[b][e, pl.ds(f * 16, 16)] * av)
            return carry

        lax.fori_loop(0, K // EU, blk, 0)

    for g in range(3):
        meta_issue(g, g)

    def zrow(i, carry):
        for f in range(D // 16):
            r0[i, pl.ds(f * 16, 16)] = jnp.zeros((16,), jnp.float32)
        return carry

    lax.fori_loop(0, K, zrow, 0)
    for i in range(ROWS_PT // K):
        pltpu.sync_copy(r0, agg_sh.at[pl.ds(s * ROWS_PT + i * K, K)])

    meta_wait(0)
    gather_issue(0)
    meta_wait(1)
    gather_issue(1)
    plsc.subcore_barrier()

    def slot(g, b, guard):
        gather_wait(b)
        scale(b)
        scatter_issue(b)
        if guard:
            @pl.when(g >= 1)
            def _():
                scatter_wait((b + 3) % NBUF)

            @pl.when(g + 3 < NCHUNK)
            def _():
                meta_issue(g + 3, (b + 3) % NBUF)

            @pl.when(g + 2 < NCHUNK)
            def _():
                meta_wait((b + 2) % NBUF)
                gather_issue((b + 2) % NBUF)
        else:
            if g >= 1:
                scatter_wait((b + 3) % NBUF)
            if g + 3 < NCHUNK:
                meta_issue(g + 3, (b + 3) % NBUF)
            if g + 2 < NCHUNK:
                meta_wait((b + 2) % NBUF)
                gather_issue((b + 2) % NBUF)

    def quad(i, carry):
        g = i * NBUF
        for b in range(NBUF):
            slot(g + b, b, True)
        return carry

    lax.fori_loop(0, (NCHUNK - 1) // NBUF, quad, 0)
    slot(NCHUNK - 1, (NCHUNK - 1) % NBUF, False)

    scatter_wait((NCHUNK - 1) % NBUF)
    plsc.subcore_barrier()
    pltpu.sync_copy(agg_sh.at[pl.ds(s * ROWS_PT, ROWS_PT)],
                    out_hbm.at[c, pl.ds(s * ROWS_PT, ROWS_PT)])


@functools.cache
def _make_sc_agg():
    mesh = plsc.VectorSubcoreMesh(
        core_axis_name="c", subcore_axis_name="s", num_cores=NC, num_subcores=NS
    )
    meta_t = [pltpu.VMEM((3, K), jnp.int32)] * NBUF
    rows_t = [pltpu.VMEM((K, D), jnp.float32)] * NBUF
    sem_t = [pltpu.SemaphoreType.DMA] * (3 * NBUF)
    return pl.kernel(
        _sc_agg_body,
        out_type=jax.ShapeDtypeStruct((NC, NPAD, D), jnp.float32),
        mesh=mesh,
        scratch_types=(meta_t + rows_t
                       + [pltpu.VMEM_SHARED((NPAD, D), jnp.float32)]
                       + sem_t),
        compiler_params=pltpu.CompilerParams(needs_layout_passes=False),
    )


def _sc_agg(m, meta):
    return _make_sc_agg()(m, meta)


BN = 1000


def _mm_body(h_ref, w_ref, o_ref):
    o_ref[...] = jnp.dot(h_ref[...], w_ref[...],
                         preferred_element_type=jnp.float32)


def _matmul(h, w):
    return pl.pallas_call(
        _mm_body,
        grid=(N // BN,),
        in_specs=[pl.BlockSpec((BN, D), lambda i: (i, 0)),
                  pl.BlockSpec((D, D), lambda i: (0, 0))],
        out_specs=pl.BlockSpec((BN, D), lambda i: (i, 0)),
        out_shape=jax.ShapeDtypeStruct((N, D), jnp.float32),
    )(h, w)


def _gru_body(p_ref, h_ref, wih_ref, whh_ref, bih_ref, bhh_ref, h_out):
    agg = p_ref[0] + p_ref[1]
    h = h_ref[...]
    gi = lax.dot_general(agg, wih_ref[...], (((1,), (1,)), ((), ())),
                         preferred_element_type=jnp.float32) + bih_ref[...]
    gh = lax.dot_general(h, whh_ref[...], (((1,), (1,)), ((), ())),
                         preferred_element_type=jnp.float32) + bhh_ref[...]
    i_r, i_z, i_n = gi[:, :D], gi[:, D:2 * D], gi[:, 2 * D:]
    h_r, h_z, h_n = gh[:, :D], gh[:, D:2 * D], gh[:, 2 * D:]
    r = jax.nn.sigmoid(i_r + h_r)
    z = jax.nn.sigmoid(i_z + h_z)
    n = jnp.tanh(i_n + r * h_n)
    h_out[...] = (1.0 - z) * n + z * h


def _gru(p, h, w_ih, w_hh, bih, bhh):
    return pl.pallas_call(
        _gru_body,
        grid=(N // BN,),
        in_specs=[
            pl.BlockSpec((NC, BN, D), lambda i: (0, i, 0)),
            pl.BlockSpec((BN, D), lambda i: (i, 0)),
            pl.BlockSpec((3 * D, D), lambda i: (0, 0)),
            pl.BlockSpec((3 * D, D), lambda i: (0, 0)),
            pl.BlockSpec((1, 3 * D), lambda i: (0, 0)),
            pl.BlockSpec((1, 3 * D), lambda i: (0, 0)),
        ],
        out_specs=pl.BlockSpec((BN, D), lambda i: (i, 0)),
        out_shape=jax.ShapeDtypeStruct((N, D), jnp.float32),
    )(p, h, w_ih, w_hh, bih, bhh)


def kernel(x, edge_index, edge_attr, weight, w_ih, w_hh, b_ih, b_hh):
    ea_bits = lax.bitcast_convert_type(edge_attr, jnp.int32)
    meta = jnp.stack(
        [edge_index[0], edge_index[1], ea_bits]
    ).reshape(3, NW, NCHUNK, K).transpose(1, 2, 0, 3)
    bih = b_ih.reshape(1, 3 * D)
    bhh = b_hh.reshape(1, 3 * D)
    h = x
    for i in range(L):
        m = _matmul(h, weight[i])
        p = _sc_agg(m, meta)
        h = _gru(p, h, w_ih, w_hh, bih, bhh)
    return h


# speedup vs baseline: 1.0315x; 1.0315x over previous
import functools

import jax
import jax.numpy as jnp
from jax import lax
from jax.experimental import pallas as pl
from jax.experimental.pallas import tpu as pltpu
from jax.experimental.pallas import tpu_sc as plsc

N = 10000
E = 320000
D = 128
L = 2

NC = 2
NS = 16
NW = NC * NS
EPW = E // NW
K = 80
NCHUNK = EPW // K
NBUF = 4
EU = 20
NPAD = 10240
ROWS_PT = NPAD // NS


def _sc_agg_body(m_hbm, src_hbm, dst_hbm, ea_hbm, out_hbm,
                 s0, s1, s2, s3, d0, d1, d2, d3, e0, e1, e2, e3,
                 r0, r1, r2, r3, agg_sh,
                 ms0, ms1, ms2, ms3, gs0, gs1, gs2, gs3,
                 ss0, ss1, ss2, ss3):
    c = lax.axis_index("c")
    s = lax.axis_index("s")
    wid = c * NS + s
    src_v = (s0, s1, s2, s3)
    dst_v = (d0, d1, d2, d3)
    ea_v = (e0, e1, e2, e3)
    rows = (r0, r1, r2, r3)
    msem = (ms0, ms1, ms2, ms3)
    gsem = (gs0, gs1, gs2, gs3)
    ssem = (ss0, ss1, ss2, ss3)

    def meta_issue(g, b):
        base = pl.multiple_of(wid * EPW + g * K, 8)
        pltpu.async_copy(src_hbm.at[pl.ds(base, K)], src_v[b], msem[b])
        pltpu.async_copy(dst_hbm.at[pl.ds(base, K)], dst_v[b], msem[b])
        pltpu.async_copy(ea_hbm.at[pl.ds(base, K)], ea_v[b], msem[b])

    def meta_wait(b):
        pltpu.make_async_copy(src_hbm.at[pl.ds(0, K)], src_v[b], msem[b]).wait()
        pltpu.make_async_copy(dst_hbm.at[pl.ds(0, K)], dst_v[b], msem[b]).wait()
        pltpu.make_async_copy(ea_hbm.at[pl.ds(0, K)], ea_v[b], msem[b]).wait()

    def gather_issue(b):
        pltpu.async_copy(m_hbm.at[src_v[b]], rows[b], gsem[b])

    def gather_wait(b):
        pltpu.make_async_copy(m_hbm.at[src_v[b]], rows[b], gsem[b]).wait()

    def scatter_issue(b):
        pltpu.async_copy(rows[b], agg_sh.at[dst_v[b]], ssem[b], add=True)

    def scatter_wait(b):
        pltpu.make_async_copy(rows[b], agg_sh.at[dst_v[b]], ssem[b]).wait()

    def scale(b):
        def blk(i, carry):
            base = i * EU
            for j in range(EU):
                e = base + j
                av = plsc.load_gather(
                    ea_v[b], [jnp.full((16,), e, dtype=jnp.int32)])
                for f in range(D // 16):
                    rows[b][e, pl.ds(f * 16, 16)] = (
                        rows[b][e, pl.ds(f * 16, 16)] * av)
            return carry

        lax.fori_loop(0, K // EU, blk, 0)

    for g in range(3):
        meta_issue(g, g)

    def zrow(i, carry):
        for f in range(D // 16):
            r0[i, pl.ds(f * 16, 16)] = jnp.zeros((16,), jnp.float32)
        return carry

    lax.fori_loop(0, K, zrow, 0)
    for i in range(ROWS_PT // K):
        pltpu.sync_copy(r0, agg_sh.at[pl.ds(s * ROWS_PT + i * K, K)])

    meta_wait(0)
    gather_issue(0)
    meta_wait(1)
    gather_issue(1)
    plsc.subcore_barrier()

    def slot(g, b, guard):
        gather_wait(b)
        scale(b)
        scatter_issue(b)
        if guard:
            @pl.when(g >= 1)
            def _():
                scatter_wait((b + 3) % NBUF)

            @pl.when(g + 3 < NCHUNK)
            def _():
                meta_issue(g + 3, (b + 3) % NBUF)

            @pl.when(g + 2 < NCHUNK)
            def _():
                meta_wait((b + 2) % NBUF)
                gather_issue((b + 2) % NBUF)
        else:
            if g >= 1:
                scatter_wait((b + 3) % NBUF)
            if g + 3 < NCHUNK:
                meta_issue(g + 3, (b + 3) % NBUF)
            if g + 2 < NCHUNK:
                meta_wait((b + 2) % NBUF)
                gather_issue((b + 2) % NBUF)

    def quad(i, carry):
        g = i * NBUF
        for b in range(NBUF):
            slot(g + b, b, True)
        return carry

    lax.fori_loop(0, (NCHUNK - 1) // NBUF, quad, 0)
    slot(NCHUNK - 1, (NCHUNK - 1) % NBUF, False)

    scatter_wait((NCHUNK - 1) % NBUF)
    plsc.subcore_barrier()
    pltpu.sync_copy(agg_sh.at[pl.ds(s * ROWS_PT, ROWS_PT)],
                    out_hbm.at[c, pl.ds(s * ROWS_PT, ROWS_PT)])


@functools.cache
def _make_sc_agg():
    mesh = plsc.VectorSubcoreMesh(
        core_axis_name="c", subcore_axis_name="s", num_cores=NC, num_subcores=NS
    )
    idx_t = [pltpu.VMEM((K,), jnp.int32)] * NBUF
    ea_t = [pltpu.VMEM((K,), jnp.float32)] * NBUF
    rows_t = [pltpu.VMEM((K, D), jnp.float32)] * NBUF
    sem_t = [pltpu.SemaphoreType.DMA] * (3 * NBUF)
    return pl.kernel(
        _sc_agg_body,
        out_type=jax.ShapeDtypeStruct((NC, NPAD, D), jnp.float32),
        mesh=mesh,
        scratch_types=(idx_t + idx_t + ea_t + rows_t
                       + [pltpu.VMEM_SHARED((NPAD, D), jnp.float32)]
                       + sem_t),
        compiler_params=pltpu.CompilerParams(needs_layout_passes=False),
    )


def _sc_agg(m, src, dst, ea):
    return _make_sc_agg()(m, src, dst, ea)


BN = 1000


def _mm_body(h_ref, w_ref, o_ref):
    o_ref[...] = jnp.dot(h_ref[...], w_ref[...],
                         preferred_element_type=jnp.float32)


def _matmul(h, w):
    return pl.pallas_call(
        _mm_body,
        grid=(N // BN,),
        in_specs=[pl.BlockSpec((BN, D), lambda i: (i, 0)),
                  pl.BlockSpec((D, D), lambda i: (0, 0))],
        out_specs=pl.BlockSpec((BN, D), lambda i: (i, 0)),
        out_shape=jax.ShapeDtypeStruct((N, D), jnp.float32),
    )(h, w)


def _gru_body(p_ref, h_ref, wih_ref, whh_ref, bih_ref, bhh_ref, h_out):
    agg = p_ref[0] + p_ref[1]
    h = h_ref[...]
    gi = lax.dot_general(agg, wih_ref[...], (((1,), (1,)), ((), ())),
                         preferred_element_type=jnp.float32) + bih_ref[...]
    gh = lax.dot_general(h, whh_ref[...], (((1,), (1,)), ((), ())),
                         preferred_element_type=jnp.float32) + bhh_ref[...]
    i_r, i_z, i_n = gi[:, :D], gi[:, D:2 * D], gi[:, 2 * D:]
    h_r, h_z, h_n = gh[:, :D], gh[:, D:2 * D], gh[:, 2 * D:]
    r = jax.nn.sigmoid(i_r + h_r)
    z = jax.nn.sigmoid(i_z + h_z)
    n = jnp.tanh(i_n + r * h_n)
    h_out[...] = (1.0 - z) * n + z * h


def _gru(p, h, w_ih, w_hh, bih, bhh):
    return pl.pallas_call(
        _gru_body,
        grid=(N // BN,),
        in_specs=[
            pl.BlockSpec((NC, BN, D), lambda i: (0, i, 0)),
            pl.BlockSpec((BN, D), lambda i: (i, 0)),
            pl.BlockSpec((3 * D, D), lambda i: (0, 0)),
            pl.BlockSpec((3 * D, D), lambda i: (0, 0)),
            pl.BlockSpec((1, 3 * D), lambda i: (0, 0)),
            pl.BlockSpec((1, 3 * D), lambda i: (0, 0)),
        ],
        out_specs=pl.BlockSpec((BN, D), lambda i: (i, 0)),
        out_shape=jax.ShapeDtypeStruct((N, D), jnp.float32),
    )(p, h, w_ih, w_hh, bih, bhh)


def kernel(x, edge_index, edge_attr, weight, w_ih, w_hh, b_ih, b_hh):
    src = edge_index[0]
    dst = edge_index[1]
    bih = b_ih.reshape(1, 3 * D)
    bhh = b_hh.reshape(1, 3 * D)
    h = x
    for i in range(L):
        m = _matmul(h, weight[i])
        p = _sc_agg(m, src, dst, edge_attr)
        h = _gru(p, h, w_ih, w_hh, bih, bhh)
    return h


# speedup vs baseline: 1.0345x; 1.0029x over previous
import functools

import jax
import jax.numpy as jnp
from jax import lax
from jax.experimental import pallas as pl
from jax.experimental.pallas import tpu as pltpu
from jax.experimental.pallas import tpu_sc as plsc

N = 10000
E = 320000
D = 128
L = 2

NC = 2
NS = 16
NW = NC * NS
EPW = E // NW
K = 80
NCHUNK = EPW // K
NBUF = 4
EU = 16
NPAD = 10240
ROWS_PT = NPAD // NS


def _sc_agg_body(m_hbm, src_hbm, dst_hbm, ea_hbm, out_hbm,
                 s0, s1, s2, s3, d0, d1, d2, d3, e0, e1, e2, e3,
                 r0, r1, r2, r3, agg_sh,
                 ms0, ms1, ms2, ms3, gs0, gs1, gs2, gs3,
                 ss0, ss1, ss2, ss3):
    c = lax.axis_index("c")
    s = lax.axis_index("s")
    wid = c * NS + s
    src_v = (s0, s1, s2, s3)
    dst_v = (d0, d1, d2, d3)
    ea_v = (e0, e1, e2, e3)
    rows = (r0, r1, r2, r3)
    msem = (ms0, ms1, ms2, ms3)
    gsem = (gs0, gs1, gs2, gs3)
    ssem = (ss0, ss1, ss2, ss3)

    def meta_issue(g, b):
        base = pl.multiple_of(wid * EPW + g * K, 8)
        pltpu.async_copy(src_hbm.at[pl.ds(base, K)], src_v[b], msem[b])
        pltpu.async_copy(dst_hbm.at[pl.ds(base, K)], dst_v[b], msem[b])
        pltpu.async_copy(ea_hbm.at[pl.ds(base, K)], ea_v[b], msem[b])

    def meta_wait(b):
        pltpu.make_async_copy(src_hbm.at[pl.ds(0, K)], src_v[b], msem[b]).wait()
        pltpu.make_async_copy(dst_hbm.at[pl.ds(0, K)], dst_v[b], msem[b]).wait()
        pltpu.make_async_copy(ea_hbm.at[pl.ds(0, K)], ea_v[b], msem[b]).wait()

    def gather_issue(b):
        pltpu.async_copy(m_hbm.at[src_v[b]], rows[b], gsem[b])

    def gather_wait(b):
        pltpu.make_async_copy(m_hbm.at[src_v[b]], rows[b], gsem[b]).wait()

    def scatter_issue(b):
        pltpu.async_copy(rows[b], agg_sh.at[dst_v[b]], ssem[b], add=True)

    def scatter_wait(b):
        pltpu.make_async_copy(rows[b], agg_sh.at[dst_v[b]], ssem[b]).wait()

    def scale(b):
        def blk(i, carry):
            base = i * EU
            for j in range(EU):
                e = base + j
                av = plsc.load_gather(
                    ea_v[b], [jnp.full((16,), e, dtype=jnp.int32)])
                for f in range(D // 16):
                    rows[b][e, pl.ds(f * 16, 16)] = (
                        rows[b][e, pl.ds(f * 16, 16)] * av)
            return carry

        lax.fori_loop(0, K // EU, blk, 0)

    for g in range(3):
        meta_issue(g, g)

    def zrow(i, carry):
        for f in range(D // 16):
            r0[i, pl.ds(f * 16, 16)] = jnp.zeros((16,), jnp.float32)
        return carry

    lax.fori_loop(0, K, zrow, 0)
    for i in range(ROWS_PT // K):
        pltpu.sync_copy(r0, agg_sh.at[pl.ds(s * ROWS_PT + i * K, K)])

    meta_wait(0)
    gather_issue(0)
    meta_wait(1)
    gather_issue(1)
    plsc.subcore_barrier()

    def slot(g, b, guard):
        gather_wait(b)
        scale(b)
        scatter_issue(b)
        if guard:
            @pl.when(g >= 1)
            def _():
                scatter_wait((b + 3) % NBUF)

            @pl.when(g + 3 < NCHUNK)
            def _():
                meta_issue(g + 3, (b + 3) % NBUF)

            @pl.when(g + 2 < NCHUNK)
            def _():
                meta_wait((b + 2) % NBUF)
                gather_issue((b + 2) % NBUF)
        else:
            if g >= 1:
                scatter_wait((b + 3) % NBUF)
            if g + 3 < NCHUNK:
                meta_issue(g + 3, (b + 3) % NBUF)
            if g + 2 < NCHUNK:
                meta_wait((b + 2) % NBUF)
                gather_issue((b + 2) % NBUF)

    def quad(i, carry):
        g = i * NBUF
        for b in range(NBUF):
            slot(g + b, b, True)
        return carry

    lax.fori_loop(0, (NCHUNK - 1) // NBUF, quad, 0)
    slot(NCHUNK - 1, (NCHUNK - 1) % NBUF, False)

    scatter_wait((NCHUNK - 1) % NBUF)
    plsc.subcore_barrier()
    pltpu.sync_copy(agg_sh.at[pl.ds(s * ROWS_PT, ROWS_PT)],
                    out_hbm.at[c, pl.ds(s * ROWS_PT, ROWS_PT)])


@functools.cache
def _make_sc_agg():
    mesh = plsc.VectorSubcoreMesh(
        core_axis_name="c", subcore_axis_name="s", num_cores=NC, num_subcores=NS
    )
    idx_t = [pltpu.VMEM((K,), jnp.int32)] * NBUF
    ea_t = [pltpu.VMEM((K,), jnp.float32)] * NBUF
    rows_t = [pltpu.VMEM((K, D), jnp.float32)] * NBUF
    sem_t = [pltpu.SemaphoreType.DMA] * (3 * NBUF)
    return pl.kernel(
        _sc_agg_body,
        out_type=jax.ShapeDtypeStruct((NC, NPAD, D), jnp.float32),
        mesh=mesh,
        scratch_types=(idx_t + idx_t + ea_t + rows_t
                       + [pltpu.VMEM_SHARED((NPAD, D), jnp.float32)]
                       + sem_t),
        compiler_params=pltpu.CompilerParams(needs_layout_passes=False),
    )


def _sc_agg(m, src, dst, ea):
    return _make_sc_agg()(m, src, dst, ea)


BN = 1000


def _mm_body(h_ref, w_ref, o_ref):
    o_ref[...] = jnp.dot(h_ref[...], w_ref[...],
                         preferred_element_type=jnp.float32)


def _matmul(h, w):
    return pl.pallas_call(
        _mm_body,
        grid=(N // BN,),
        in_specs=[pl.BlockSpec((BN, D), lambda i: (i, 0)),
                  pl.BlockSpec((D, D), lambda i: (0, 0))],
        out_specs=pl.BlockSpec((BN, D), lambda i: (i, 0)),
        out_shape=jax.ShapeDtypeStruct((N, D), jnp.float32),
    )(h, w)


def _gru_body(p_ref, h_ref, wih_ref, whh_ref, bih_ref, bhh_ref, h_out):
    agg = p_ref[0] + p_ref[1]
    h = h_ref[...]
    gi = lax.dot_general(agg, wih_ref[...], (((1,), (1,)), ((), ())),
                         preferred_element_type=jnp.float32) + bih_ref[...]
    gh = lax.dot_general(h, whh_ref[...], (((1,), (1,)), ((), ())),
                         preferred_element_type=jnp.float32) + bhh_ref[...]
    i_r, i_z, i_n = gi[:, :D], gi[:, D:2 * D], gi[:, 2 * D:]
    h_r, h_z, h_n = gh[:, :D], gh[:, D:2 * D], gh[:, 2 * D:]
    r = jax.nn.sigmoid(i_r + h_r)
    z = jax.nn.sigmoid(i_z + h_z)
    n = jnp.tanh(i_n + r * h_n)
    h_out[...] = (1.0 - z) * n + z * h


def _gru(p, h, w_ih, w_hh, bih, bhh):
    return pl.pallas_call(
        _gru_body,
        grid=(N // BN,),
        in_specs=[
            pl.BlockSpec((NC, BN, D), lambda i: (0, i, 0)),
            pl.BlockSpec((BN, D), lambda i: (i, 0)),
            pl.BlockSpec((3 * D, D), lambda i: (0, 0)),
            pl.BlockSpec((3 * D, D), lambda i: (0, 0)),
            pl.BlockSpec((1, 3 * D), lambda i: (0, 0)),
            pl.BlockSpec((1, 3 * D), lambda i: (0, 0)),
        ],
        out_specs=pl.BlockSpec((BN, D), lambda i: (i, 0)),
        out_shape=jax.ShapeDtypeStruct((N, D), jnp.float32),
    )(p, h, w_ih, w_hh, bih, bhh)


def kernel(x, edge_index, edge_attr, weight, w_ih, w_hh, b_ih, b_hh):
    src = edge_index[0]
    dst = edge_index[1]
    bih = b_ih.reshape(1, 3 * D)
    bhh = b_hh.reshape(1, 3 * D)
    h = x
    for i in range(L):
        m = _matmul(h, weight[i])
        p = _sc_agg(m, src, dst, edge_attr)
        h = _gru(p, h, w_ih, w_hh, bih, bhh)
    return h
